# baseline (speedup 1.0000x reference)
import jax
import jax.numpy as jnp
from jax import lax
from jax.experimental import pallas as pl
from jax.experimental.pallas import tpu as pltpu
from jax.experimental.pallas import tpu_sc as plsc

N = 10000
D = 256
H = 128
NC = 2
NT = 16
CH = 128
NCH = 79
EPT = NCH * CH
EPAD = NT * EPT
NPAD = 10112
STRIPE = NPAD // NT
SCHUNKS = ((0, 128), (128, 128), (256, 128), (384, 128), (512, 120))
RB = 1000
GRID = N // RB


def _prep_edges(ei):
    e = ei.shape[1]
    pad = EPAD - e
    spread = jnp.arange(pad, dtype=ei.dtype)
    src = jnp.concatenate([ei[0], spread % N])
    dst = jnp.concatenate([ei[1], N + spread % (NPAD - N)])
    return src.reshape(NT, NCH, CH), dst.reshape(NT, NCH, CH)


def _tc_pre_body(x_ref, wp_ref, wc_ref, asp_ref, adp_ref, asc_ref, adc_ref,
                 xwp_o, xwc_o, asp_o, adp_o, asc_o, adc_o):
    x = x_ref[...]
    xwp = jnp.dot(x, wp_ref[...], preferred_element_type=jnp.float32)
    xwc = jnp.dot(x, wc_ref[...], preferred_element_type=jnp.float32)
    xwp_o[0] = xwp[:, :H]
    xwp_o[1] = xwp[:, H:]
    xwc_o[0] = xwc[:, :H]
    xwc_o[1] = xwc[:, H:]
    asp_o[...] = jnp.broadcast_to(
        jnp.sum(xwp * asp_ref[...], axis=1, keepdims=True), (RB, 8))
    adp_o[...] = jnp.broadcast_to(
        jnp.sum(xwp * adp_ref[...], axis=1, keepdims=True), (RB, 8))
    asc_o[...] = jnp.broadcast_to(
        jnp.sum(xwc * asc_ref[...], axis=1, keepdims=True), (RB, 8))
    adc_o[...] = jnp.broadcast_to(
        jnp.sum(xwc * adc_ref[...], axis=1, keepdims=True), (RB, 8))


def _tc_pre(x, wp, wc, att_sp, att_dp, att_sc, att_dc):
    row = pl.BlockSpec((RB, D), lambda i: (i, 0))
    halves = pl.BlockSpec((NC, RB, H), lambda i: (0, i, 0))
    sca = pl.BlockSpec((RB, 8), lambda i: (i, 0))
    full = pl.BlockSpec((D, D), lambda i: (0, 0))
    vec = pl.BlockSpec((1, D), lambda i: (0, 0))
    return pl.pallas_call(
        _tc_pre_body,
        grid=(GRID,),
        in_specs=[row, full, full, vec, vec, vec, vec],
        out_specs=[halves, halves, sca, sca, sca, sca],
        out_shape=[jax.ShapeDtypeStruct((NC, N, H), jnp.float32)] * 2
        + [jax.ShapeDtypeStruct((N, 8), jnp.float32)] * 4,
    )(x, wp, wc, att_sp.reshape(1, D), att_dp.reshape(1, D),
      att_sc.reshape(1, D), att_dc.reshape(1, D))


def _sc_body(srcp, dstp, srcc, dstc, srcr, dstr,
             xwp2, xwc2, x2,
             asp, adp, asc, adc,
             o12, mean,
             acc, den, cnt,
             src_v, dst_v, ex_v, rows_v, asb, adb, cntb,
             semA, semB, semC, semD):
    cid = lax.axis_index("c")
    sid = lax.axis_index("s")
    roff = cid * N

    def zero_rows():
        def zr(i, carry):
            for k in range(8):
                rows_v[i, pl.ds(k * 16, 16)] = jnp.zeros((16,), jnp.float32)
            return carry
        lax.fori_loop(0, CH, zr, 0)

    def zero_acc_stripe():
        base = sid * STRIPE
        for off, sz in SCHUNKS:
            pltpu.sync_copy(rows_v.at[pl.ds(0, sz)],
                            acc.at[pl.ds(base + off, sz)])

    def zero_vec_stripe(ref):
        for k in range(8):
            asb[pl.ds(k * 16, 16)] = jnp.zeros((16,), jnp.float32)
        base = sid * STRIPE
        for off, sz in SCHUNKS:
            pltpu.sync_copy(asb.at[pl.ds(0, sz)],
                            ref.at[pl.ds(base + off, sz)])

    def offset_src():
        def poff(c, carry):
            for k in range(8):
                sl = pl.ds(k * 16, 16)
                src_v[c, sl] = src_v[c, sl] + roff
            return carry
        lax.fori_loop(0, NCH, poff, 0)

    def scale_half(c, h):
        def sg(g, carry):
            cfv = ex_v[c, pl.ds(h * 64 + g * 16, 16)]
            for j in range(16):
                cf = cfv[j]
                r = h * 64 + g * 16 + j
                for k in range(8):
                    sl = pl.ds(k * 16, 16)
                    rows_v[r, sl] = rows_v[r, sl] * cf
            return carry
        lax.fori_loop(0, 4, sg, 0)

    def gat(src3, dst3, as_h, ad_h, tbl):
        pltpu.sync_copy(src3.at[sid], src_v)
        pltpu.sync_copy(dst3.at[sid], dst_v)
        plsc.subcore_barrier()

        def expchunk(c, b):
            for k in range(8):
                sl = pl.ds(k * 16, 16)
                sb = pl.ds(b * CH + k * 16, 16)
                al = asb[sb] + adb[sb]
                al = jnp.where(al >= 0.0, al, al * 0.2)
                ex_v[c, sl] = jnp.exp(al)

        def l1(i, carry):
            c0 = i * 2
            c1 = c0 + 1
            g0a = pltpu.async_copy(as_h.at[src_v.at[c0]],
                                   asb.at[pl.ds(0, CH)], semA)
            g0b = pltpu.async_copy(ad_h.at[dst_v.at[c0]],
                                   adb.at[pl.ds(0, CH)], semA)
            g1a = pltpu.async_copy(as_h.at[src_v.at[c1]],
                                   asb.at[pl.ds(CH, CH)], semB)
            g1b = pltpu.async_copy(ad_h.at[dst_v.at[c1]],
                                   adb.at[pl.ds(CH, CH)], semB)
            g0a.wait()
            g0b.wait()
            expchunk(c0, 0)
            s0 = pltpu.async_copy(ex_v.at[c0], den.at[dst_v.at[c0]],
                                  semC, add=True)
            g1a.wait()
            g1b.wait()
            expchunk(c1, 1)
            s1 = pltpu.async_copy(ex_v.at[c1], den.at[dst_v.at[c1]],
                                  semD, add=True)
            s0.wait()
            s1.wait()
            return carry
        lax.fori_loop(0, NCH // 2, l1, 0)
        cl = NCH - 1
        pltpu.async_copy(as_h.at[src_v.at[cl]],
                         asb.at[pl.ds(0, CH)], semA).wait()
        pltpu.async_copy(ad_h.at[dst_v.at[cl]],
                         adb.at[pl.ds(0, CH)], semA).wait()
        expchunk(cl, 0)
        pltpu.sync_copy(ex_v.at[cl], den.at[dst_v.at[cl]], add=True)

        plsc.subcore_barrier()
        offset_src()

        def rowchunk(c, gd, dhalf):
            g0 = pltpu.async_copy(tbl.at[src_v.at[c, pl.ds(0, 64)]],
                                  rows_v.at[pl.ds(0, 64)], semA)
            g1 = pltpu.async_copy(tbl.at[src_v.at[c, pl.ds(64, 64)]],
                                  rows_v.at[pl.ds(64, 64)], semB)
            gd.wait()
            for k in range(8):
                sl = pl.ds(k * 16, 16)
                sb = pl.ds(dhalf * CH + k * 16, 16)
                ex_v[c, sl] = ex_v[c, sl] / (asb[sb] + 1e-16)
            g0.wait()
            scale_half(c, 0)
            g1.wait()
            scale_half(c, 1)
            s = pltpu.async_copy(rows_v, acc.at[dst_v.at[c]], semC, add=True)
            return s

        def l2(i, carry):
            c0 = i * 2
            c1 = c0 + 1
            gd0 = pltpu.async_copy(den.at[dst_v.at[c0]],
                                   asb.at[pl.ds(0, CH)], semD)
            s0 = rowchunk(c0, gd0, 0)
            gd1 = pltpu.async_copy(den.at[dst_v.at[c1]],
                                   asb.at[pl.ds(CH, CH)], semD)
            s0.wait()
            s1 = rowchunk(c1, gd1, 1)
            s1.wait()
            return carry
        lax.fori_loop(0, NCH // 2, l2, 0)
        gdl = pltpu.async_copy(den.at[dst_v.at[cl]],
                               asb.at[pl.ds(0, CH)], semD)
        sl_ = rowchunk(cl, gdl, 0)
        sl_.wait()

        plsc.subcore_barrier()
        zero_vec_stripe(den)

    def sage():
        pltpu.sync_copy(srcr.at[sid], src_v)
        pltpu.sync_copy(dstr.at[sid], dst_v)
        offset_src()
        for k in range(8):
            ex_v[0, pl.ds(k * 16, 16)] = jnp.full((16,), 1.0, jnp.float32)

        def p4(c, carry):
            pltpu.async_copy(x2.at[src_v.at[c]], rows_v, semA).wait()
            s = pltpu.async_copy(rows_v, acc.at[dst_v.at[c]], semB, add=True)
            pltpu.sync_copy(ex_v.at[0], cnt.at[dst_v.at[c]], add=True)
            s.wait()
            return carry
        lax.fori_loop(0, NCH, p4, 0)

    def write_acc(dst_hbm):
        base = sid * STRIPE
        for off, sz in SCHUNKS:
            pltpu.sync_copy(acc.at[pl.ds(base + off, sz)],
                            dst_hbm.at[cid, pl.ds(base + off, sz)])

    def write_mean():
        base = sid * STRIPE
        pltpu.sync_copy(cnt.at[pl.ds(base, STRIPE)],
                        cntb.at[pl.ds(0, STRIPE)])

        def pinv(g, carry):
            sl = pl.ds(g * 16, 16)
            cntb[sl] = 1.0 / jnp.maximum(cntb[sl], 1.0)
            return carry
        lax.fori_loop(0, (STRIPE + 15) // 16, pinv, 0)

        for off, sz in SCHUNKS:
            pltpu.sync_copy(acc.at[pl.ds(base + off, sz)],
                            rows_v.at[pl.ds(0, sz)])

            def mgrp(g, carry):
                cfv = cntb[pl.ds(off + g * 16, 16)]
                for jj in range(16):
                    cf = cfv[jj]
                    r = g * 16 + jj
                    for k in range(8):
                        sl = pl.ds(k * 16, 16)
                        rows_v[r, sl] = rows_v[r, sl] * cf
                return carry
            lax.fori_loop(0, CH // 16, mgrp, 0)
            pltpu.sync_copy(rows_v.at[pl.ds(0, sz)],
                            mean.at[cid, pl.ds(base + off, sz)])

    zero_rows()
    zero_acc_stripe()
    zero_vec_stripe(den)
    zero_vec_stripe(cnt)
    plsc.subcore_barrier()

    gat(srcp, dstp, asp, adp, xwp2)
    gat(srcc, dstc, asc, adc, xwc2)

    plsc.subcore_barrier()
    write_acc(o12)
    plsc.subcore_barrier()
    zero_rows()
    zero_acc_stripe()
    plsc.subcore_barrier()

    sage()
    plsc.subcore_barrier()
    write_mean()


def _sc_aggregate(srcp, dstp, srcc, dstc, srcr, dstr,
                  xwp2, xwc2, x2,
                  asp, adp, asc, adc):
    mesh = plsc.VectorSubcoreMesh(core_axis_name="c", subcore_axis_name="s")
    out_type = (jax.ShapeDtypeStruct((NC, NPAD, H), jnp.float32),
                jax.ShapeDtypeStruct((NC, NPAD, H), jnp.float32))
    scratch = [
        pltpu.VMEM_SHARED((NPAD, H), jnp.float32),
        pltpu.VMEM_SHARED((NPAD,), jnp.float32),
        pltpu.VMEM_SHARED((NPAD,), jnp.float32),
        pltpu.VMEM((NCH, CH), jnp.int32),
        pltpu.VMEM((NCH, CH), jnp.int32),
        pltpu.VMEM((NCH, CH), jnp.float32),
        pltpu.VMEM((CH, H), jnp.float32),
        pltpu.VMEM((2 * CH,), jnp.float32),
        pltpu.VMEM((2 * CH,), jnp.float32),
        pltpu.VMEM((640,), jnp.float32),
        pltpu.SemaphoreType.DMA,
        pltpu.SemaphoreType.DMA,
        pltpu.SemaphoreType.DMA,
        pltpu.SemaphoreType.DMA,
    ]
    return pl.kernel(
        _sc_body, out_type=out_type, mesh=mesh, scratch_types=scratch,
        compiler_params=pltpu.CompilerParams(needs_layout_passes=False),
    )(srcp, dstp, srcc, dstc, srcr, dstr,
      xwp2, xwc2, x2,
      asp, adp, asc, adc)


def _tc_post_body(o12_ref, mean_ref, x_ref, wl_ref, wr_ref, b_ref, out_ref):
    wl = wl_ref[...]
    agg = (jnp.dot(mean_ref[0], wl[:H], preferred_element_type=jnp.float32)
           + jnp.dot(mean_ref[1], wl[H:], preferred_element_type=jnp.float32))
    root = jnp.dot(x_ref[...], wr_ref[...], preferred_element_type=jnp.float32)
    o12 = jnp.concatenate([o12_ref[0], o12_ref[1]], axis=1)
    out_ref[...] = (o12 + agg + root + b_ref[...]) / 3.0


def _tc_post(o12, mean, x, wl, wr, bsum):
    pair = pl.BlockSpec((NC, RB, H), lambda i: (0, i, 0))
    row = pl.BlockSpec((RB, D), lambda i: (i, 0))
    full = pl.BlockSpec((D, D), lambda i: (0, 0))
    vec = pl.BlockSpec((1, D), lambda i: (0, 0))
    return pl.pallas_call(
        _tc_post_body,
        grid=(GRID,),
        in_specs=[pair, pair, row, full, full, vec],
        out_specs=row,
        out_shape=jax.ShapeDtypeStruct((N, D), jnp.float32),
    )(o12, mean, x, wl, wr, bsum.reshape(1, D))


def kernel(skill_embed, edge_index_parent, edge_index_child, edge_index_relate,
           W_parent, att_src_parent, att_dst_parent, bias_parent,
           W_child, att_src_child, att_dst_child, bias_child,
           W_l_relate, b_l_relate, W_r_relate):
    x = skill_embed

    (xwp3, xwc3, asp8, adp8, asc8, adc8) = _tc_pre(
        x, W_parent, W_child,
        att_src_parent, att_dst_parent, att_src_child, att_dst_child)

    pad = NPAD - N
    asp = jnp.pad(asp8[:, 0], (0, pad))
    adp = jnp.pad(adp8[:, 0], (0, pad))
    asc = jnp.pad(asc8[:, 0], (0, pad))
    adc = jnp.pad(adc8[:, 0], (0, pad))

    xwp2 = xwp3.reshape(NC * N, H)
    xwc2 = xwc3.reshape(NC * N, H)
    x2 = jnp.concatenate([x[:, :H], x[:, H:]], axis=0)

    srcp, dstp = _prep_edges(edge_index_parent)
    srcc, dstc = _prep_edges(edge_index_child)
    srcr, dstr = _prep_edges(edge_index_relate)

    o12, mean = _sc_aggregate(srcp, dstp, srcc, dstc, srcr, dstr,
                              xwp2, xwc2, x2,
                              asp, adp, asc, adc)

    bsum = bias_parent + bias_child + b_l_relate
    return _tc_post(o12, mean, x, W_l_relate, W_r_relate, bsum)


# speedup vs baseline: 1.0376x; 1.0376x over previous
import jax
import jax.numpy as jnp
from jax import lax
from jax.experimental import pallas as pl
from jax.experimental.pallas import tpu as pltpu
from jax.experimental.pallas import tpu_sc as plsc

N = 10000
D = 256
H = 128
NC = 2
NT = 16
CH = 128
NCH = 79
EPT = NCH * CH
EPAD = NT * EPT
NPAD = 10112
STRIPE = NPAD // NT
SCHUNKS = ((0, 128), (128, 128), (256, 128), (384, 128), (512, 120))
RB = 1000
GRID = N // RB


def _prep_edges(ei):
    e = ei.shape[1]
    pad = EPAD - e
    spread = jnp.arange(pad, dtype=ei.dtype)
    src = jnp.concatenate([ei[0], spread % N])
    dst = jnp.concatenate([ei[1], N + spread % (NPAD - N)])
    return src.reshape(NT, NCH, CH), dst.reshape(NT, NCH, CH)


def _tc_pre_body(x_ref, wp_ref, wc_ref, asp_ref, adp_ref, asc_ref, adc_ref,
                 xwp_o, xwc_o, x_o, asp_o, adp_o, asc_o, adc_o):
    x = x_ref[...]
    xwp = jnp.dot(x, wp_ref[...], preferred_element_type=jnp.float32)
    xwc = jnp.dot(x, wc_ref[...], preferred_element_type=jnp.float32)
    xwp_o[0] = xwp[:, :H]
    xwp_o[1] = xwp[:, H:]
    xwc_o[0] = xwc[:, :H]
    xwc_o[1] = xwc[:, H:]
    x_o[0] = x[:, :H]
    x_o[1] = x[:, H:]
    asp_o[...] = jnp.broadcast_to(
        jnp.sum(xwp * asp_ref[...], axis=1, keepdims=True), (RB, 8))
    adp_o[...] = jnp.broadcast_to(
        jnp.sum(xwp * adp_ref[...], axis=1, keepdims=True), (RB, 8))
    asc_o[...] = jnp.broadcast_to(
        jnp.sum(xwc * asc_ref[...], axis=1, keepdims=True), (RB, 8))
    adc_o[...] = jnp.broadcast_to(
        jnp.sum(xwc * adc_ref[...], axis=1, keepdims=True), (RB, 8))


def _tc_pre(x, wp, wc, att_sp, att_dp, att_sc, att_dc):
    row = pl.BlockSpec((RB, D), lambda i: (i, 0))
    halves = pl.BlockSpec((NC, RB, H), lambda i: (0, i, 0))
    sca = pl.BlockSpec((RB, 8), lambda i: (i, 0))
    full = pl.BlockSpec((D, D), lambda i: (0, 0))
    vec = pl.BlockSpec((1, D), lambda i: (0, 0))
    return pl.pallas_call(
        _tc_pre_body,
        grid=(GRID,),
        in_specs=[row, full, full, vec, vec, vec, vec],
        out_specs=[halves, halves, halves, sca, sca, sca, sca],
        out_shape=[jax.ShapeDtypeStruct((NC, N, H), jnp.float32)] * 3
        + [jax.ShapeDtypeStruct((N, 8), jnp.float32)] * 4,
    )(x, wp, wc, att_sp.reshape(1, D), att_dp.reshape(1, D),
      att_sc.reshape(1, D), att_dc.reshape(1, D))


def _sc_body(srcp, dstp, srcc, dstc, srcr, dstr,
             xwp2, xwc2, x2,
             asp, adp, asc, adc,
             o12, mean,
             acc, den, cnt,
             src_v, dst_v, ex_v, rows_v, asb, adb, cntb,
             semA, semB, semC, semD):
    cid = lax.axis_index("c")
    sid = lax.axis_index("s")
    roff = cid * N

    def zero_rows():
        def zr(i, carry):
            for k in range(8):
                rows_v[i, pl.ds(k * 16, 16)] = jnp.zeros((16,), jnp.float32)
            return carry
        lax.fori_loop(0, CH, zr, 0)

    def zero_acc_stripe():
        base = sid * STRIPE
        for off, sz in SCHUNKS:
            pltpu.sync_copy(rows_v.at[pl.ds(0, sz)],
                            acc.at[pl.ds(base + off, sz)])

    def zero_vec_stripe(ref):
        for k in range(8):
            asb[pl.ds(k * 16, 16)] = jnp.zeros((16,), jnp.float32)
        base = sid * STRIPE
        for off, sz in SCHUNKS:
            pltpu.sync_copy(asb.at[pl.ds(0, sz)],
                            ref.at[pl.ds(base + off, sz)])

    def offset_src():
        def poff(c, carry):
            for k in range(8):
                sl = pl.ds(k * 16, 16)
                src_v[c, sl] = src_v[c, sl] + roff
            return carry
        lax.fori_loop(0, NCH, poff, 0)

    def scale_half(c, h):
        def sg(g, carry):
            cfv = ex_v[c, pl.ds(h * 64 + g * 16, 16)]
            for j in range(16):
                cf = cfv[j]
                r = h * 64 + g * 16 + j
                for k in range(8):
                    sl = pl.ds(k * 16, 16)
                    rows_v[r, sl] = rows_v[r, sl] * cf
            return carry
        lax.fori_loop(0, 4, sg, 0)

    def gat(src3, dst3, as_h, ad_h, tbl):
        pltpu.sync_copy(src3.at[sid], src_v)
        pltpu.sync_copy(dst3.at[sid], dst_v)
        plsc.subcore_barrier()

        def expchunk(c, b):
            for k in range(8):
                sl = pl.ds(k * 16, 16)
                sb = pl.ds(b * CH + k * 16, 16)
                al = asb[sb] + adb[sb]
                al = jnp.where(al >= 0.0, al, al * 0.2)
                ex_v[c, sl] = jnp.exp(al)

        def l1(i, carry):
            c0 = i * 2
            c1 = c0 + 1
            g0a = pltpu.async_copy(as_h.at[src_v.at[c0]],
                                   asb.at[pl.ds(0, CH)], semA)
            g0b = pltpu.async_copy(ad_h.at[dst_v.at[c0]],
                                   adb.at[pl.ds(0, CH)], semA)
            g1a = pltpu.async_copy(as_h.at[src_v.at[c1]],
                                   asb.at[pl.ds(CH, CH)], semB)
            g1b = pltpu.async_copy(ad_h.at[dst_v.at[c1]],
                                   adb.at[pl.ds(CH, CH)], semB)
            g0a.wait()
            g0b.wait()
            expchunk(c0, 0)
            s0 = pltpu.async_copy(ex_v.at[c0], den.at[dst_v.at[c0]],
                                  semC, add=True)
            g1a.wait()
            g1b.wait()
            expchunk(c1, 1)
            s1 = pltpu.async_copy(ex_v.at[c1], den.at[dst_v.at[c1]],
                                  semD, add=True)
            s0.wait()
            s1.wait()
            return carry
        lax.fori_loop(0, NCH // 2, l1, 0)
        cl = NCH - 1
        pltpu.async_copy(as_h.at[src_v.at[cl]],
                         asb.at[pl.ds(0, CH)], semA).wait()
        pltpu.async_copy(ad_h.at[dst_v.at[cl]],
                         adb.at[pl.ds(0, CH)], semA).wait()
        expchunk(cl, 0)
        pltpu.sync_copy(ex_v.at[cl], den.at[dst_v.at[cl]], add=True)

        plsc.subcore_barrier()
        offset_src()

        def rowchunk(c, gd, dhalf):
            g0 = pltpu.async_copy(tbl.at[src_v.at[c, pl.ds(0, 64)]],
                                  rows_v.at[pl.ds(0, 64)], semA)
            g1 = pltpu.async_copy(tbl.at[src_v.at[c, pl.ds(64, 64)]],
                                  rows_v.at[pl.ds(64, 64)], semB)
            gd.wait()
            for k in range(8):
                sl = pl.ds(k * 16, 16)
                sb = pl.ds(dhalf * CH + k * 16, 16)
                ex_v[c, sl] = ex_v[c, sl] / (asb[sb] + 1e-16)
            g0.wait()
            scale_half(c, 0)
            g1.wait()
            scale_half(c, 1)
            s = pltpu.async_copy(rows_v, acc.at[dst_v.at[c]], semC, add=True)
            return s

        def l2(i, carry):
            c0 = i * 2
            c1 = c0 + 1
            gd0 = pltpu.async_copy(den.at[dst_v.at[c0]],
                                   asb.at[pl.ds(0, CH)], semD)
            s0 = rowchunk(c0, gd0, 0)
            gd1 = pltpu.async_copy(den.at[dst_v.at[c1]],
                                   asb.at[pl.ds(CH, CH)], semD)
            s0.wait()
            s1 = rowchunk(c1, gd1, 1)
            s1.wait()
            return carry
        lax.fori_loop(0, NCH // 2, l2, 0)
        gdl = pltpu.async_copy(den.at[dst_v.at[cl]],
                               asb.at[pl.ds(0, CH)], semD)
        sl_ = rowchunk(cl, gdl, 0)
        sl_.wait()

        plsc.subcore_barrier()
        zero_vec_stripe(den)

    def sage():
        pltpu.sync_copy(srcr.at[sid], src_v)
        pltpu.sync_copy(dstr.at[sid], dst_v)
        offset_src()
        for k in range(8):
            ex_v[0, pl.ds(k * 16, 16)] = jnp.full((16,), 1.0, jnp.float32)

        def p4(c, carry):
            pltpu.async_copy(x2.at[src_v.at[c]], rows_v, semA).wait()
            s = pltpu.async_copy(rows_v, acc.at[dst_v.at[c]], semB, add=True)
            pltpu.sync_copy(ex_v.at[0], cnt.at[dst_v.at[c]], add=True)
            s.wait()
            return carry
        lax.fori_loop(0, NCH, p4, 0)

    def write_acc(dst_hbm):
        base = sid * STRIPE
        for off, sz in SCHUNKS:
            pltpu.sync_copy(acc.at[pl.ds(base + off, sz)],
                            dst_hbm.at[cid, pl.ds(base + off, sz)])

    def write_mean():
        base = sid * STRIPE
        pltpu.sync_copy(cnt.at[pl.ds(base, STRIPE)],
                        cntb.at[pl.ds(0, STRIPE)])

        def pinv(g, carry):
            sl = pl.ds(g * 16, 16)
            cntb[sl] = 1.0 / jnp.maximum(cntb[sl], 1.0)
            return carry
        lax.fori_loop(0, (STRIPE + 15) // 16, pinv, 0)

        for off, sz in SCHUNKS:
            pltpu.sync_copy(acc.at[pl.ds(base + off, sz)],
                            rows_v.at[pl.ds(0, sz)])

            def mgrp(g, carry):
                cfv = cntb[pl.ds(off + g * 16, 16)]
                for jj in range(16):
                    cf = cfv[jj]
                    r = g * 16 + jj
                    for k in range(8):
                        sl = pl.ds(k * 16, 16)
                        rows_v[r, sl] = rows_v[r, sl] * cf
                return carry
            lax.fori_loop(0, CH // 16, mgrp, 0)
            pltpu.sync_copy(rows_v.at[pl.ds(0, sz)],
                            mean.at[cid, pl.ds(base + off, sz)])

    zero_rows()
    zero_acc_stripe()
    zero_vec_stripe(den)
    zero_vec_stripe(cnt)
    plsc.subcore_barrier()

    gat(srcp, dstp, asp, adp, xwp2)
    gat(srcc, dstc, asc, adc, xwc2)

    plsc.subcore_barrier()
    write_acc(o12)
    plsc.subcore_barrier()
    zero_rows()
    zero_acc_stripe()
    plsc.subcore_barrier()

    sage()
    plsc.subcore_barrier()
    write_mean()


def _sc_aggregate(srcp, dstp, srcc, dstc, srcr, dstr,
                  xwp2, xwc2, x2,
                  asp, adp, asc, adc):
    mesh = plsc.VectorSubcoreMesh(core_axis_name="c", subcore_axis_name="s")
    out_type = (jax.ShapeDtypeStruct((NC, NPAD, H), jnp.float32),
                jax.ShapeDtypeStruct((NC, NPAD, H), jnp.float32))
    scratch = [
        pltpu.VMEM_SHARED((NPAD, H), jnp.float32),
        pltpu.VMEM_SHARED((NPAD,), jnp.float32),
        pltpu.VMEM_SHARED((NPAD,), jnp.float32),
        pltpu.VMEM((NCH, CH), jnp.int32),
        pltpu.VMEM((NCH, CH), jnp.int32),
        pltpu.VMEM((NCH, CH), jnp.float32),
        pltpu.VMEM((CH, H), jnp.float32),
        pltpu.VMEM((2 * CH,), jnp.float32),
        pltpu.VMEM((2 * CH,), jnp.float32),
        pltpu.VMEM((640,), jnp.float32),
        pltpu.SemaphoreType.DMA,
        pltpu.SemaphoreType.DMA,
        pltpu.SemaphoreType.DMA,
        pltpu.SemaphoreType.DMA,
    ]
    return pl.kernel(
        _sc_body, out_type=out_type, mesh=mesh, scratch_types=scratch,
        compiler_params=pltpu.CompilerParams(needs_layout_passes=False),
    )(srcp, dstp, srcc, dstc, srcr, dstr,
      xwp2, xwc2, x2,
      asp, adp, asc, adc)


def _tc_post_body(o12_ref, mean_ref, x_ref, wl_ref, wr_ref, b_ref, out_ref):
    wl = wl_ref[...]
    agg = (jnp.dot(mean_ref[0], wl[:H], preferred_element_type=jnp.float32)
           + jnp.dot(mean_ref[1], wl[H:], preferred_element_type=jnp.float32))
    root = jnp.dot(x_ref[...], wr_ref[...], preferred_element_type=jnp.float32)
    o12 = jnp.concatenate([o12_ref[0], o12_ref[1]], axis=1)
    out_ref[...] = (o12 + agg + root + b_ref[...]) / 3.0


def _tc_post(o12, mean, x, wl, wr, bsum):
    pair = pl.BlockSpec((NC, RB, H), lambda i: (0, i, 0))
    row = pl.BlockSpec((RB, D), lambda i: (i, 0))
    full = pl.BlockSpec((D, D), lambda i: (0, 0))
    vec = pl.BlockSpec((1, D), lambda i: (0, 0))
    return pl.pallas_call(
        _tc_post_body,
        grid=(GRID,),
        in_specs=[pair, pair, row, full, full, vec],
        out_specs=row,
        out_shape=jax.ShapeDtypeStruct((N, D), jnp.float32),
    )(o12, mean, x, wl, wr, bsum.reshape(1, D))


def kernel(skill_embed, edge_index_parent, edge_index_child, edge_index_relate,
           W_parent, att_src_parent, att_dst_parent, bias_parent,
           W_child, att_src_child, att_dst_child, bias_child,
           W_l_relate, b_l_relate, W_r_relate):
    x = skill_embed

    (xwp3, xwc3, x3, asp8, adp8, asc8, adc8) = _tc_pre(
        x, W_parent, W_child,
        att_src_parent, att_dst_parent, att_src_child, att_dst_child)

    pad = NPAD - N
    asp = jnp.pad(asp8[:, 0], (0, pad))
    adp = jnp.pad(adp8[:, 0], (0, pad))
    asc = jnp.pad(asc8[:, 0], (0, pad))
    adc = jnp.pad(adc8[:, 0], (0, pad))

    xwp2 = xwp3.reshape(NC * N, H)
    xwc2 = xwc3.reshape(NC * N, H)
    x2 = x3.reshape(NC * N, H)

    srcp, dstp = _prep_edges(edge_index_parent)
    srcc, dstc = _prep_edges(edge_index_child)
    srcr, dstr = _prep_edges(edge_index_relate)

    o12, mean = _sc_aggregate(srcp, dstp, srcc, dstc, srcr, dstr,
                              xwp2, xwc2, x2,
                              asp, adp, asc, adc)

    bsum = bias_parent + bias_child + b_l_relate
    return _tc_post(o12, mean, x, W_l_relate, W_r_relate, bsum)
